# baseline (speedup 1.0000x reference)
import functools

import numpy as np
import jax
import jax.numpy as jnp
from jax import lax
from jax.experimental import pallas as pl
from jax.experimental.pallas import tpu as pltpu
from jax.experimental.pallas import tpu_sc as plsc

_N = 8192
_SD = 64
_CD = 32
_OD = 128
_H = 512
_ALPHA = 0.5
_ROWS = 8
_CHUNK = 1024
_TINY = float(np.finfo(np.float32).tiny)
_PREC = lax.Precision.HIGHEST

_NC = 2
_NS = 16
_NW = _NC * _NS
_BPW = _N // _NW


def _tf_mix(x0, x1, r):
    x0 = x0 + x1
    x1 = ((x1 << r) | (x1 >> (32 - r))) ^ x0
    return x0, x1


def _threefry_0_7(n_u32):
    ks1 = jnp.uint32(7)
    ks2 = jnp.uint32(0x1BD11BDD)
    x1 = n_u32 + ks1
    x0 = x1
    x1 = ((x1 << 13) | (x1 >> 19)) ^ x0
    for r in (15, 26, 6):
        x0, x1 = _tf_mix(x0, x1, r)
    x0 = x0 + ks1
    x1 = x1 + jnp.uint32(0x1BD11BDE)
    for r in (17, 29, 16, 24):
        x0, x1 = _tf_mix(x0, x1, r)
    x0 = x0 + ks2
    x1 = x1 + jnp.uint32(2)
    for r in (13, 15, 26, 6):
        x0, x1 = _tf_mix(x0, x1, r)
    x1 = x1 + jnp.uint32(10)
    for r in (17, 29, 16, 24):
        x0, x1 = _tf_mix(x0, x1, r)
    x0 = x0 + ks1
    x1 = x1 + jnp.uint32(0x1BD11BE1)
    for r in (13, 15, 26, 6):
        x0, x1 = _tf_mix(x0, x1, r)
    x0 = x0 + ks2
    x1 = x1 + jnp.uint32(5)
    return x0 ^ x1


def _mlp_body(sp_ref, lw_ref, noise_ref, obs_ref, ctrl_ref, dW1_ref,
              db1_ref, dW2_ref, db2_ref, mW1_ref, mb1_ref, mW2_ref,
              mb2_ref, spred_ref, lwp_ref):
    sp = sp_ref[...]
    rb = sp.shape[0]
    ctrl_b = jnp.broadcast_to(ctrl_ref[...], (rb, _CD))
    h = jnp.tanh(jnp.dot(jnp.concatenate([sp, ctrl_b], axis=1),
                         dW1_ref[...]) + db1_ref[...])
    delta = jnp.dot(h, dW2_ref[...]) + db2_ref[...]
    spred = sp + delta + noise_ref[...]
    obs_b = jnp.broadcast_to(obs_ref[...], (rb, _OD))
    hm = jnp.tanh(jnp.dot(jnp.concatenate([obs_b, spred], axis=1),
                          mW1_ref[...]) + mb1_ref[...])
    ll = jnp.dot(hm, mW2_ref[...]) + mb2_ref[0, 0]
    spred_ref[...] = jnp.concatenate(
        [spred, jnp.zeros_like(spred)], axis=1)
    lwp_ref[...] = lw_ref[...] + ll


def _finalize_body(lwp_ref, spred_ref, logw_ref, negc_ref, best_ref):
    lwp = lwp_ref[...]
    w = _ALPHA * jnp.exp(lwp) + (1.0 - _ALPHA) / _N
    negc_ref[...] = -1.0 / w
    lw1 = lwp - jnp.log(w)
    amax = jnp.max(lw1)
    ls = jnp.log(jnp.sum(jnp.exp(lw1 - amax))) + amax
    logw_ref[...] = lw1 - ls
    gm = jnp.max(lwp)
    fi = (lax.broadcasted_iota(jnp.int32, (8, 1024), 0) * 1024
          + lax.broadcasted_iota(jnp.int32, (8, 1024), 1))
    bi = jnp.min(jnp.where(lwp == gm, fi, jnp.int32(_N)))
    fcol = lax.broadcasted_iota(jnp.int32, (_N, 1), 0)
    msk = (fcol == bi).astype(jnp.float32)
    best_ref[...] = jnp.sum(spred_ref[...] * msk, axis=0, keepdims=True)


@functools.cache
def _sc_gather_kernel():
    @functools.partial(
        pl.kernel,
        mesh=plsc.VectorSubcoreMesh(core_axis_name="c",
                                    subcore_axis_name="s"),
        out_type=jax.ShapeDtypeStruct((_N, 2 * _SD), jnp.float32),
        scratch_types=[
            pltpu.VMEM((2, 128), jnp.int32),
            pltpu.VMEM((_BPW, 2 * _SD), jnp.float32),
            pltpu.SemaphoreType.DMA,
        ],
    )
    def _sc_gather(table_hbm, idx_hbm, out_hbm, idx_v, rows_v, sem):
        wid = lax.axis_index("s") * _NC + lax.axis_index("c")
        base = wid * _BPW
        pltpu.sync_copy(idx_hbm.at[pl.ds(wid * 2, 2)], idx_v)
        cps = [
            pltpu.async_copy(table_hbm.at[idx_v.at[j]],
                             rows_v.at[pl.ds(j * 128, 128)], sem)
            for j in range(2)
        ]
        for cp in cps:
            cp.wait()
        pltpu.sync_copy(rows_v, out_hbm.at[pl.ds(base, _BPW)])

    return _sc_gather


def _sample_body(negc_ref, idx_ref):
    p = pl.program_id(0)
    di = lax.broadcasted_iota(jnp.int32, (_ROWS, _CHUNK), 0)
    jj = lax.broadcasted_iota(jnp.int32, (_ROWS, _CHUNK), 1)
    base = (p * _ROWS + di) * _N + jj
    vmin = jnp.full((_ROWS, _CHUNK), jnp.inf, dtype=jnp.float32)
    varg = jnp.zeros((_ROWS, _CHUNK), dtype=jnp.int32)
    for cc in range(_N // _CHUNK):
        n = (base + cc * _CHUNK).astype(jnp.uint32)
        bits = _threefry_0_7(n)
        fb = (bits >> jnp.uint32(9)) | jnp.uint32(0x3F800000)
        u = jnp.maximum(
            lax.bitcast_convert_type(fb, jnp.float32) - 1.0, _TINY)
        met = jnp.log(u) * negc_ref[cc:cc + 1, :]
        upd = met < vmin
        vmin = jnp.where(upd, met, vmin)
        varg = jnp.where(upd, cc * _CHUNK + jj, varg)
    rowmin = jnp.min(vmin, axis=1, keepdims=True)
    cand = jnp.where(vmin == rowmin, varg, jnp.int32(_N))
    idx_ref[...] = jnp.min(cand, axis=1, keepdims=True).reshape(1, _ROWS, 1)


def _mlp_call(sp, lw, noise, obs_r, ctrl_r, dW1, db1, dW2, db2, mW1, mb1,
              mW2, mb2):
    nb = 16
    rb = _N // nb
    full = lambda shape: pl.BlockSpec(shape, lambda i: (0, 0))
    return pl.pallas_call(
        _mlp_body,
        grid=(nb,),
        in_specs=[
            pl.BlockSpec((rb, _SD), lambda i: (i, 0)),
            pl.BlockSpec((rb, 1), lambda i: (i, 0)),
            pl.BlockSpec((rb, _SD), lambda i: (i, 0)),
            full((1, _OD)),
            full((1, _CD)),
            full((_SD + _CD, _H)),
            full((1, _H)),
            full((_H, _SD)),
            full((1, _SD)),
            full((_OD + _SD, _H)),
            full((1, _H)),
            full((_H, 1)),
            pl.BlockSpec(memory_space=pltpu.SMEM),
        ],
        out_specs=[
            pl.BlockSpec((rb, 2 * _SD), lambda i: (i, 0)),
            pl.BlockSpec((rb, 1), lambda i: (i, 0)),
        ],
        out_shape=[
            jax.ShapeDtypeStruct((_N, 2 * _SD), jnp.float32),
            jax.ShapeDtypeStruct((_N, 1), jnp.float32),
        ],
    )(sp, lw, noise, obs_r, ctrl_r, dW1, db1, dW2, db2, mW1, mb1, mW2,
      mb2)


def _finalize_call(lwp2, spred):
    return pl.pallas_call(
        _finalize_body,
        out_shape=[
            jax.ShapeDtypeStruct((8, 1024), jnp.float32),
            jax.ShapeDtypeStruct((8, 1024), jnp.float32),
            jax.ShapeDtypeStruct((1, 2 * _SD), jnp.float32),
        ],
    )(lwp2, spred)


def _sample_call(negc):
    ng = _N // _ROWS
    return pl.pallas_call(
        _sample_body,
        grid=(ng,),
        in_specs=[pl.BlockSpec((8, 1024), lambda i: (0, 0))],
        out_specs=pl.BlockSpec((1, _ROWS, 1), lambda i: (i, 0, 0)),
        out_shape=jax.ShapeDtypeStruct((ng, _ROWS, 1), jnp.int32),
    )(negc)


def kernel(states_prev, log_weights_prev, observation, control, dyn_W1,
           dyn_b1, dyn_W2, dyn_b2, meas_W1, meas_b1, meas_W2, meas_b2):
    noise = 0.1 * jax.random.normal(jax.random.key(42), (_N, _SD),
                                    dtype=jnp.float32)
    spred, lwp = _mlp_call(
        states_prev, log_weights_prev.reshape(_N, 1), noise,
        observation.reshape(1, _OD), control.reshape(1, _CD),
        dyn_W1, dyn_b1.reshape(1, _H), dyn_W2, dyn_b2.reshape(1, _SD),
        meas_W1, meas_b1.reshape(1, _H), meas_W2,
        meas_b2.reshape(1, 1))
    logw, negc, best = _finalize_call(lwp.reshape(8, 1024), spred)
    indices = _sample_call(negc).reshape(64, 128)
    states = _sc_gather_kernel()(spred, indices)
    return best[0, :_SD], states[:, :_SD], logw.reshape(_N)


# speedup vs baseline: 1.1714x; 1.1714x over previous
import functools

import numpy as np
import jax
import jax.numpy as jnp
from jax import lax
from jax.experimental import pallas as pl
from jax.experimental.pallas import tpu as pltpu
from jax.experimental.pallas import tpu_sc as plsc

_N = 8192
_SD = 64
_CD = 32
_OD = 128
_H = 512
_ALPHA = 0.5
_ROWS = 32
_CHUNK = 1024
_LN2 = float(np.log(2.0))
_TINY = float(np.finfo(np.float32).tiny)
_PREC = lax.Precision.HIGHEST

_NC = 2
_NS = 16
_NW = _NC * _NS
_BPW = _N // _NW


def _tf_mix(x0, x1, r):
    x0 = x0 + x1
    x1 = ((x1 << r) | (x1 >> (32 - r))) ^ x0
    return x0, x1


def _threefry_0_7(n_u32):
    ks1 = jnp.uint32(7)
    ks2 = jnp.uint32(0x1BD11BDD)
    x1 = n_u32 + ks1
    x0 = x1
    x1 = ((x1 << 13) | (x1 >> 19)) ^ x0
    for r in (15, 26, 6):
        x0, x1 = _tf_mix(x0, x1, r)
    x0 = x0 + ks1
    x1 = x1 + jnp.uint32(0x1BD11BDE)
    for r in (17, 29, 16, 24):
        x0, x1 = _tf_mix(x0, x1, r)
    x0 = x0 + ks2
    x1 = x1 + jnp.uint32(2)
    for r in (13, 15, 26, 6):
        x0, x1 = _tf_mix(x0, x1, r)
    x1 = x1 + jnp.uint32(10)
    for r in (17, 29, 16, 24):
        x0, x1 = _tf_mix(x0, x1, r)
    x0 = x0 + ks1
    x1 = x1 + jnp.uint32(0x1BD11BE1)
    for r in (13, 15, 26, 6):
        x0, x1 = _tf_mix(x0, x1, r)
    x0 = x0 + ks2
    x1 = x1 + jnp.uint32(5)
    return x0 ^ x1


def _mlp_body(sp_ref, lw_ref, noise_ref, obs_ref, ctrl_ref, dW1_ref,
              db1_ref, dW2_ref, db2_ref, mW1_ref, mb1_ref, mW2_ref,
              mb2_ref, spred_ref, lwp_ref):
    sp = sp_ref[...]
    rb = sp.shape[0]
    ctrl_b = jnp.broadcast_to(ctrl_ref[...], (rb, _CD))
    h = jnp.tanh(jnp.dot(jnp.concatenate([sp, ctrl_b], axis=1),
                         dW1_ref[...]) + db1_ref[...])
    delta = jnp.dot(h, dW2_ref[...]) + db2_ref[...]
    spred = sp + delta + noise_ref[...]
    obs_b = jnp.broadcast_to(obs_ref[...], (rb, _OD))
    hm = jnp.tanh(jnp.dot(jnp.concatenate([obs_b, spred], axis=1),
                          mW1_ref[...]) + mb1_ref[...])
    ll = jnp.dot(hm, mW2_ref[...]) + mb2_ref[0, 0]
    spred_ref[...] = jnp.concatenate(
        [spred, jnp.zeros_like(spred)], axis=1)
    lwp_ref[...] = lw_ref[...] + ll


def _finalize_body(lwp_ref, spred_ref, logw_ref, negc_ref, best_ref):
    lwp = lwp_ref[...]
    w = _ALPHA * jnp.exp(lwp) + (1.0 - _ALPHA) / _N
    negc_ref[...] = -1.0 / w
    lw1 = lwp - jnp.log(w)
    amax = jnp.max(lw1)
    ls = jnp.log(jnp.sum(jnp.exp(lw1 - amax))) + amax
    logw_ref[...] = lw1 - ls
    gm = jnp.max(lwp)
    fi = (lax.broadcasted_iota(jnp.int32, (8, 1024), 0) * 1024
          + lax.broadcasted_iota(jnp.int32, (8, 1024), 1))
    bi = jnp.min(jnp.where(lwp == gm, fi, jnp.int32(_N)))
    fcol = lax.broadcasted_iota(jnp.int32, (_N, 1), 0)
    msk = (fcol == bi).astype(jnp.float32)
    best_ref[...] = jnp.sum(spred_ref[...] * msk, axis=0, keepdims=True)


@functools.cache
def _sc_gather_kernel():
    @functools.partial(
        pl.kernel,
        mesh=plsc.VectorSubcoreMesh(core_axis_name="c",
                                    subcore_axis_name="s"),
        out_type=jax.ShapeDtypeStruct((_N, 2 * _SD), jnp.float32),
        scratch_types=[
            pltpu.VMEM((2, 128), jnp.int32),
            pltpu.VMEM((_BPW, 2 * _SD), jnp.float32),
            pltpu.SemaphoreType.DMA,
        ],
    )
    def _sc_gather(table_hbm, idx_hbm, out_hbm, idx_v, rows_v, sem):
        wid = lax.axis_index("s") * _NC + lax.axis_index("c")
        base = wid * _BPW
        pltpu.sync_copy(idx_hbm.at[pl.ds(wid * 2, 2)], idx_v)
        cps = [
            pltpu.async_copy(table_hbm.at[idx_v.at[j]],
                             rows_v.at[pl.ds(j * 128, 128)], sem)
            for j in range(2)
        ]
        for cp in cps:
            cp.wait()
        pltpu.sync_copy(rows_v, out_hbm.at[pl.ds(base, _BPW)])

    return _sc_gather


def _sample_body(negc_ref, idx_ref):
    p = pl.program_id(0)
    di = lax.broadcasted_iota(jnp.int32, (_ROWS, _CHUNK), 0)
    jj = lax.broadcasted_iota(jnp.int32, (_ROWS, _CHUNK), 1)
    base = (p * _ROWS + di) * _N + jj
    vmin = jnp.full((_ROWS, _CHUNK), jnp.inf, dtype=jnp.float32)
    varg = jnp.zeros((_ROWS, _CHUNK), dtype=jnp.int32)
    for cc in range(_N // _CHUNK):
        n = (base + cc * _CHUNK).astype(jnp.uint32)
        bits = _threefry_0_7(n)
        fb = (bits >> jnp.uint32(9)) | jnp.uint32(0x3F800000)
        u = jnp.maximum(
            lax.bitcast_convert_type(fb, jnp.float32) - 1.0, _TINY)
        met = jnp.log(u) * negc_ref[cc:cc + 1, :]
        upd = met < vmin
        vmin = jnp.where(upd, met, vmin)
        varg = jnp.where(upd, cc * _CHUNK + jj, varg)
    rowmin = jnp.min(vmin, axis=1, keepdims=True)
    cand = jnp.where(vmin == rowmin, varg, jnp.int32(_N))
    idx_ref[...] = jnp.min(cand, axis=1, keepdims=True).reshape(1, _ROWS, 1)


def _mlp_call(sp, lw, noise, obs_r, ctrl_r, dW1, db1, dW2, db2, mW1, mb1,
              mW2, mb2):
    nb = 16
    rb = _N // nb
    full = lambda shape: pl.BlockSpec(shape, lambda i: (0, 0))
    return pl.pallas_call(
        _mlp_body,
        grid=(nb,),
        in_specs=[
            pl.BlockSpec((rb, _SD), lambda i: (i, 0)),
            pl.BlockSpec((rb, 1), lambda i: (i, 0)),
            pl.BlockSpec((rb, _SD), lambda i: (i, 0)),
            full((1, _OD)),
            full((1, _CD)),
            full((_SD + _CD, _H)),
            full((1, _H)),
            full((_H, _SD)),
            full((1, _SD)),
            full((_OD + _SD, _H)),
            full((1, _H)),
            full((_H, 1)),
            pl.BlockSpec(memory_space=pltpu.SMEM),
        ],
        out_specs=[
            pl.BlockSpec((rb, 2 * _SD), lambda i: (i, 0)),
            pl.BlockSpec((rb, 1), lambda i: (i, 0)),
        ],
        out_shape=[
            jax.ShapeDtypeStruct((_N, 2 * _SD), jnp.float32),
            jax.ShapeDtypeStruct((_N, 1), jnp.float32),
        ],
    )(sp, lw, noise, obs_r, ctrl_r, dW1, db1, dW2, db2, mW1, mb1, mW2,
      mb2)


def _finalize_call(lwp2, spred):
    return pl.pallas_call(
        _finalize_body,
        out_shape=[
            jax.ShapeDtypeStruct((8, 1024), jnp.float32),
            jax.ShapeDtypeStruct((8, 1024), jnp.float32),
            jax.ShapeDtypeStruct((1, 2 * _SD), jnp.float32),
        ],
    )(lwp2, spred)


def _sample_call(negc):
    ng = _N // _ROWS
    return pl.pallas_call(
        _sample_body,
        grid=(ng,),
        in_specs=[pl.BlockSpec((8, 1024), lambda i: (0, 0))],
        out_specs=pl.BlockSpec((1, _ROWS, 1), lambda i: (i, 0, 0)),
        out_shape=jax.ShapeDtypeStruct((ng, _ROWS, 1), jnp.int32),
    )(negc)


def kernel(states_prev, log_weights_prev, observation, control, dyn_W1,
           dyn_b1, dyn_W2, dyn_b2, meas_W1, meas_b1, meas_W2, meas_b2):
    noise = 0.1 * jax.random.normal(jax.random.key(42), (_N, _SD),
                                    dtype=jnp.float32)
    spred, lwp = _mlp_call(
        states_prev, log_weights_prev.reshape(_N, 1), noise,
        observation.reshape(1, _OD), control.reshape(1, _CD),
        dyn_W1, dyn_b1.reshape(1, _H), dyn_W2, dyn_b2.reshape(1, _SD),
        meas_W1, meas_b1.reshape(1, _H), meas_W2,
        meas_b2.reshape(1, 1))
    logw, negc, best = _finalize_call(lwp.reshape(8, 1024), spred)
    indices = _sample_call(negc).reshape(64, 128)
    states = _sc_gather_kernel()(spred, indices)
    return best[0, :_SD], states[:, :_SD], logw.reshape(_N)


# speedup vs baseline: 1.1956x; 1.0207x over previous
import functools

import numpy as np
import jax
import jax.numpy as jnp
from jax import lax
from jax.experimental import pallas as pl
from jax.experimental.pallas import tpu as pltpu
from jax.experimental.pallas import tpu_sc as plsc

_N = 8192
_SD = 64
_CD = 32
_OD = 128
_H = 512
_ALPHA = 0.5
_ROWS = 64
_CHUNK = 1024
_LN2 = float(np.log(2.0))
_TINY = float(np.finfo(np.float32).tiny)
_PREC = lax.Precision.HIGHEST

_NC = 2
_NS = 16
_NW = _NC * _NS
_BPW = _N // _NW


def _tf_mix(x0, x1, r):
    x0 = x0 + x1
    x1 = ((x1 << r) | (x1 >> (32 - r))) ^ x0
    return x0, x1


def _threefry_0_7(n_u32):
    ks1 = jnp.uint32(7)
    ks2 = jnp.uint32(0x1BD11BDD)
    x1 = n_u32 + ks1
    x0 = x1
    x1 = ((x1 << 13) | (x1 >> 19)) ^ x0
    for r in (15, 26, 6):
        x0, x1 = _tf_mix(x0, x1, r)
    x0 = x0 + ks1
    x1 = x1 + jnp.uint32(0x1BD11BDE)
    for r in (17, 29, 16, 24):
        x0, x1 = _tf_mix(x0, x1, r)
    x0 = x0 + ks2
    x1 = x1 + jnp.uint32(2)
    for r in (13, 15, 26, 6):
        x0, x1 = _tf_mix(x0, x1, r)
    x1 = x1 + jnp.uint32(10)
    for r in (17, 29, 16, 24):
        x0, x1 = _tf_mix(x0, x1, r)
    x0 = x0 + ks1
    x1 = x1 + jnp.uint32(0x1BD11BE1)
    for r in (13, 15, 26, 6):
        x0, x1 = _tf_mix(x0, x1, r)
    x0 = x0 + ks2
    x1 = x1 + jnp.uint32(5)
    return x0 ^ x1


def _mlp_body(sp_ref, lw_ref, noise_ref, obs_ref, ctrl_ref, dW1_ref,
              db1_ref, dW2_ref, db2_ref, mW1_ref, mb1_ref, mW2_ref,
              mb2_ref, spred_ref, lwp_ref):
    sp = sp_ref[...]
    rb = sp.shape[0]
    ctrl_b = jnp.broadcast_to(ctrl_ref[...], (rb, _CD))
    h = jnp.tanh(jnp.dot(jnp.concatenate([sp, ctrl_b], axis=1),
                         dW1_ref[...]) + db1_ref[...])
    delta = jnp.dot(h, dW2_ref[...]) + db2_ref[...]
    spred = sp + delta + noise_ref[...]
    obs_b = jnp.broadcast_to(obs_ref[...], (rb, _OD))
    hm = jnp.tanh(jnp.dot(jnp.concatenate([obs_b, spred], axis=1),
                          mW1_ref[...]) + mb1_ref[...])
    ll = jnp.dot(hm, mW2_ref[...]) + mb2_ref[0, 0]
    spred_ref[...] = jnp.concatenate(
        [spred, jnp.zeros_like(spred)], axis=1)
    lwp_ref[...] = lw_ref[...] + ll


def _finalize_body(lwp_ref, spred_ref, logw_ref, negc_ref, best_ref):
    lwp = lwp_ref[...]
    w = _ALPHA * jnp.exp(lwp) + (1.0 - _ALPHA) / _N
    negc_ref[...] = -1.0 / w
    lw1 = lwp - jnp.log(w)
    amax = jnp.max(lw1)
    ls = jnp.log(jnp.sum(jnp.exp(lw1 - amax))) + amax
    logw_ref[...] = lw1 - ls
    gm = jnp.max(lwp)
    fi = (lax.broadcasted_iota(jnp.int32, (8, 1024), 0) * 1024
          + lax.broadcasted_iota(jnp.int32, (8, 1024), 1))
    bi = jnp.min(jnp.where(lwp == gm, fi, jnp.int32(_N)))
    fcol = lax.broadcasted_iota(jnp.int32, (_N, 1), 0)
    msk = (fcol == bi).astype(jnp.float32)
    best_ref[...] = jnp.sum(spred_ref[...] * msk, axis=0, keepdims=True)


@functools.cache
def _sc_gather_kernel():
    @functools.partial(
        pl.kernel,
        mesh=plsc.VectorSubcoreMesh(core_axis_name="c",
                                    subcore_axis_name="s"),
        out_type=jax.ShapeDtypeStruct((_N, 2 * _SD), jnp.float32),
        scratch_types=[
            pltpu.VMEM((2, 128), jnp.int32),
            pltpu.VMEM((_BPW, 2 * _SD), jnp.float32),
            pltpu.SemaphoreType.DMA,
        ],
    )
    def _sc_gather(table_hbm, idx_hbm, out_hbm, idx_v, rows_v, sem):
        wid = lax.axis_index("s") * _NC + lax.axis_index("c")
        base = wid * _BPW
        pltpu.sync_copy(idx_hbm.at[pl.ds(wid * 2, 2)], idx_v)
        cps = [
            pltpu.async_copy(table_hbm.at[idx_v.at[j]],
                             rows_v.at[pl.ds(j * 128, 128)], sem)
            for j in range(2)
        ]
        for cp in cps:
            cp.wait()
        pltpu.sync_copy(rows_v, out_hbm.at[pl.ds(base, _BPW)])

    return _sc_gather


def _sample_body(negc_ref, idx_ref):
    p = pl.program_id(0)
    di = lax.broadcasted_iota(jnp.int32, (_ROWS, _CHUNK), 0)
    jj = lax.broadcasted_iota(jnp.int32, (_ROWS, _CHUNK), 1)
    base = (p * _ROWS + di) * _N + jj
    vmin = jnp.full((_ROWS, _CHUNK), jnp.inf, dtype=jnp.float32)
    varg = jnp.zeros((_ROWS, _CHUNK), dtype=jnp.int32)
    for cc in range(_N // _CHUNK):
        n = (base + cc * _CHUNK).astype(jnp.uint32)
        bits = _threefry_0_7(n)
        fb = (bits >> jnp.uint32(9)) | jnp.uint32(0x3F800000)
        u = jnp.maximum(
            lax.bitcast_convert_type(fb, jnp.float32) - 1.0, _TINY)
        met = jnp.log(u) * negc_ref[cc:cc + 1, :]
        upd = met < vmin
        vmin = jnp.where(upd, met, vmin)
        varg = jnp.where(upd, cc * _CHUNK + jj, varg)
    rowmin = jnp.min(vmin, axis=1, keepdims=True)
    cand = jnp.where(vmin == rowmin, varg, jnp.int32(_N))
    idx_ref[...] = jnp.min(cand, axis=1, keepdims=True).reshape(1, _ROWS, 1)


def _mlp_call(sp, lw, noise, obs_r, ctrl_r, dW1, db1, dW2, db2, mW1, mb1,
              mW2, mb2):
    nb = 16
    rb = _N // nb
    full = lambda shape: pl.BlockSpec(shape, lambda i: (0, 0))
    return pl.pallas_call(
        _mlp_body,
        grid=(nb,),
        in_specs=[
            pl.BlockSpec((rb, _SD), lambda i: (i, 0)),
            pl.BlockSpec((rb, 1), lambda i: (i, 0)),
            pl.BlockSpec((rb, _SD), lambda i: (i, 0)),
            full((1, _OD)),
            full((1, _CD)),
            full((_SD + _CD, _H)),
            full((1, _H)),
            full((_H, _SD)),
            full((1, _SD)),
            full((_OD + _SD, _H)),
            full((1, _H)),
            full((_H, 1)),
            pl.BlockSpec(memory_space=pltpu.SMEM),
        ],
        out_specs=[
            pl.BlockSpec((rb, 2 * _SD), lambda i: (i, 0)),
            pl.BlockSpec((rb, 1), lambda i: (i, 0)),
        ],
        out_shape=[
            jax.ShapeDtypeStruct((_N, 2 * _SD), jnp.float32),
            jax.ShapeDtypeStruct((_N, 1), jnp.float32),
        ],
    )(sp, lw, noise, obs_r, ctrl_r, dW1, db1, dW2, db2, mW1, mb1, mW2,
      mb2)


def _finalize_call(lwp2, spred):
    return pl.pallas_call(
        _finalize_body,
        out_shape=[
            jax.ShapeDtypeStruct((8, 1024), jnp.float32),
            jax.ShapeDtypeStruct((8, 1024), jnp.float32),
            jax.ShapeDtypeStruct((1, 2 * _SD), jnp.float32),
        ],
    )(lwp2, spred)


def _sample_call(negc):
    ng = _N // _ROWS
    return pl.pallas_call(
        _sample_body,
        grid=(ng,),
        in_specs=[pl.BlockSpec((8, 1024), lambda i: (0, 0))],
        out_specs=pl.BlockSpec((1, _ROWS, 1), lambda i: (i, 0, 0)),
        out_shape=jax.ShapeDtypeStruct((ng, _ROWS, 1), jnp.int32),
    )(negc)


def kernel(states_prev, log_weights_prev, observation, control, dyn_W1,
           dyn_b1, dyn_W2, dyn_b2, meas_W1, meas_b1, meas_W2, meas_b2):
    noise = 0.1 * jax.random.normal(jax.random.key(42), (_N, _SD),
                                    dtype=jnp.float32)
    spred, lwp = _mlp_call(
        states_prev, log_weights_prev.reshape(_N, 1), noise,
        observation.reshape(1, _OD), control.reshape(1, _CD),
        dyn_W1, dyn_b1.reshape(1, _H), dyn_W2, dyn_b2.reshape(1, _SD),
        meas_W1, meas_b1.reshape(1, _H), meas_W2,
        meas_b2.reshape(1, 1))
    logw, negc, best = _finalize_call(lwp.reshape(8, 1024), spred)
    indices = _sample_call(negc).reshape(64, 128)
    states = _sc_gather_kernel()(spred, indices)
    return best[0, :_SD], states[:, :_SD], logw.reshape(_N)


# speedup vs baseline: 1.4502x; 1.2130x over previous
import functools

import numpy as np
import jax
import jax.numpy as jnp
from jax import lax
from jax.experimental import pallas as pl
from jax.experimental.pallas import tpu as pltpu
from jax.experimental.pallas import tpu_sc as plsc

_N = 8192
_SD = 64
_CD = 32
_OD = 128
_H = 512
_ALPHA = 0.5
_ROWS = 64
_CHUNK = 1024
_LN2 = float(np.log(2.0))
_TINY = float(np.finfo(np.float32).tiny)
_PREC = lax.Precision.HIGHEST

_NC = 2
_NS = 16
_NW = _NC * _NS
_BPW = _N // _NW

_RSC = 1792
_RPW = _RSC // _NW


def _tf_mix(x0, x1, r):
    x0 = x0 + x1
    x1 = ((x1 << r) | (x1 >> (32 - r))) ^ x0
    return x0, x1


def _threefry_0_7(n_u32):
    ks1 = jnp.uint32(7)
    ks2 = jnp.uint32(0x1BD11BDD)
    x1 = n_u32 + ks1
    x0 = x1
    x1 = ((x1 << 13) | (x1 >> 19)) ^ x0
    for r in (15, 26, 6):
        x0, x1 = _tf_mix(x0, x1, r)
    x0 = x0 + ks1
    x1 = x1 + jnp.uint32(0x1BD11BDE)
    for r in (17, 29, 16, 24):
        x0, x1 = _tf_mix(x0, x1, r)
    x0 = x0 + ks2
    x1 = x1 + jnp.uint32(2)
    for r in (13, 15, 26, 6):
        x0, x1 = _tf_mix(x0, x1, r)
    x1 = x1 + jnp.uint32(10)
    for r in (17, 29, 16, 24):
        x0, x1 = _tf_mix(x0, x1, r)
    x0 = x0 + ks1
    x1 = x1 + jnp.uint32(0x1BD11BE1)
    for r in (13, 15, 26, 6):
        x0, x1 = _tf_mix(x0, x1, r)
    x0 = x0 + ks2
    x1 = x1 + jnp.uint32(5)
    return x0 ^ x1


def _mlp_body(sp_ref, lw_ref, noise_ref, obs_ref, ctrl_ref, dW1_ref,
              db1_ref, dW2_ref, db2_ref, mW1_ref, mb1_ref, mW2_ref,
              mb2_ref, spred_ref, lwp_ref):
    sp = sp_ref[...]
    rb = sp.shape[0]
    ctrl_b = jnp.broadcast_to(ctrl_ref[...], (rb, _CD))
    h = jnp.tanh(jnp.dot(jnp.concatenate([sp, ctrl_b], axis=1),
                         dW1_ref[...]) + db1_ref[...])
    delta = jnp.dot(h, dW2_ref[...]) + db2_ref[...]
    spred = sp + delta + noise_ref[...]
    obs_b = jnp.broadcast_to(obs_ref[...], (rb, _OD))
    hm = jnp.tanh(jnp.dot(jnp.concatenate([obs_b, spred], axis=1),
                          mW1_ref[...]) + mb1_ref[...])
    ll = jnp.dot(hm, mW2_ref[...]) + mb2_ref[0, 0]
    spred_ref[...] = jnp.concatenate(
        [spred, jnp.zeros_like(spred)], axis=1)
    lwp_ref[...] = lw_ref[...] + ll


def _finalize_body(lwp_ref, spred_ref, logw_ref, negc_ref, best_ref):
    lwp = lwp_ref[...]
    w = _ALPHA * jnp.exp(lwp) + (1.0 - _ALPHA) / _N
    negc_ref[...] = -1.0 / w
    lw1 = lwp - jnp.log(w)
    amax = jnp.max(lw1)
    ls = jnp.log(jnp.sum(jnp.exp(lw1 - amax))) + amax
    logw_ref[...] = lw1 - ls
    gm = jnp.max(lwp)
    fi = (lax.broadcasted_iota(jnp.int32, (8, 1024), 0) * 1024
          + lax.broadcasted_iota(jnp.int32, (8, 1024), 1))
    bi = jnp.min(jnp.where(lwp == gm, fi, jnp.int32(_N)))
    fcol = lax.broadcasted_iota(jnp.int32, (_N, 1), 0)
    msk = (fcol == bi).astype(jnp.float32)
    best_ref[...] = jnp.sum(spred_ref[...] * msk, axis=0, keepdims=True)


@functools.cache
def _sc_gather_kernel():
    @functools.partial(
        pl.kernel,
        mesh=plsc.VectorSubcoreMesh(core_axis_name="c",
                                    subcore_axis_name="s"),
        out_type=jax.ShapeDtypeStruct((_N, 2 * _SD), jnp.float32),
        scratch_types=[
            pltpu.VMEM((2, 128), jnp.int32),
            pltpu.VMEM((_BPW, 2 * _SD), jnp.float32),
            pltpu.SemaphoreType.DMA,
        ],
    )
    def _sc_gather(table_hbm, idx_hbm, out_hbm, idx_v, rows_v, sem):
        wid = lax.axis_index("s") * _NC + lax.axis_index("c")
        base = wid * _BPW
        pltpu.sync_copy(idx_hbm.at[pl.ds(wid * 2, 2)], idx_v)
        cps = [
            pltpu.async_copy(table_hbm.at[idx_v.at[j]],
                             rows_v.at[pl.ds(j * 128, 128)], sem)
            for j in range(2)
        ]
        for cp in cps:
            cp.wait()
        pltpu.sync_copy(rows_v, out_hbm.at[pl.ds(base, _BPW)])

    return _sc_gather


def _argmin_update(met, jbase, jj, vmin, varg):
    upd = met < vmin
    vmin = jnp.where(upd, met, vmin)
    varg = jnp.where(upd, jbase + jj, varg)
    return vmin, varg


def _argmin_epilogue(vmin, varg, idx_ref):
    rowmin = jnp.min(vmin, axis=1, keepdims=True)
    cand = jnp.where(vmin == rowmin, varg, jnp.int32(_N))
    idx_ref[...] = jnp.min(cand, axis=1, keepdims=True).reshape(1, _ROWS, 1)


def _bits_to_met(bits, negc_chunk):
    fb = (bits >> jnp.uint32(9)) | jnp.uint32(0x3F800000)
    u = lax.bitcast_convert_type(fb, jnp.float32) - 1.0
    return jnp.log(u) * negc_chunk


def _sample_body(negc_ref, idx_ref):
    p = pl.program_id(0)
    di = lax.broadcasted_iota(jnp.int32, (_ROWS, _CHUNK), 0)
    jj = lax.broadcasted_iota(jnp.int32, (_ROWS, _CHUNK), 1)
    base = (_RSC + p * _ROWS + di) * _N + jj
    vmin = jnp.full((_ROWS, _CHUNK), jnp.inf, dtype=jnp.float32)
    varg = jnp.zeros((_ROWS, _CHUNK), dtype=jnp.int32)
    for cc in range(_N // _CHUNK):
        n = (base + cc * _CHUNK).astype(jnp.uint32)
        met = _bits_to_met(_threefry_0_7(n), negc_ref[cc:cc + 1, :])
        vmin, varg = _argmin_update(met, cc * _CHUNK, jj, vmin, varg)
    _argmin_epilogue(vmin, varg, idx_ref)


def _sample_light_body(bits_ref, negc_ref, idx_ref):
    jj = lax.broadcasted_iota(jnp.int32, (_ROWS, _CHUNK), 1)
    vmin = jnp.full((_ROWS, _CHUNK), jnp.inf, dtype=jnp.float32)
    varg = jnp.zeros((_ROWS, _CHUNK), dtype=jnp.int32)
    for cc in range(_N // _CHUNK):
        bits = bits_ref[:, cc * _CHUNK:(cc + 1) * _CHUNK]
        met = _bits_to_met(bits, negc_ref[cc:cc + 1, :])
        vmin, varg = _argmin_update(met, cc * _CHUNK, jj, vmin, varg)
    _argmin_epilogue(vmin, varg, idx_ref)


@functools.cache
def _sc_bits_kernel():
    @functools.partial(
        pl.kernel,
        mesh=plsc.VectorSubcoreMesh(core_axis_name="c",
                                    subcore_axis_name="s"),
        out_type=jax.ShapeDtypeStruct((_RSC, _N), jnp.uint32),
        scratch_types=[
            pltpu.VMEM((_N,), jnp.uint32),
        ],
    )
    def _sc_bits(out_hbm, buf):
        wid = lax.axis_index("s") * _NC + lax.axis_index("c")
        base_row = wid * _RPW
        lane = lax.iota(jnp.int32, 16)

        def row_body(r, carry):
            row = base_row + r

            def ck_body(ck, carry2):
                for q in range(8):
                    col = ck * 128 + q * 16
                    n = (row * _N + col + lane).astype(jnp.uint32)
                    buf[pl.ds(col, 16)] = _threefry_0_7(n)
                return carry2

            lax.fori_loop(0, _N // 128, ck_body, 0)
            pltpu.sync_copy(buf, out_hbm.at[row])
            return carry

        lax.fori_loop(0, _RPW, row_body, 0)

    return _sc_bits


def _mlp_call(sp, lw, noise, obs_r, ctrl_r, dW1, db1, dW2, db2, mW1, mb1,
              mW2, mb2):
    nb = 16
    rb = _N // nb
    full = lambda shape: pl.BlockSpec(shape, lambda i: (0, 0))
    return pl.pallas_call(
        _mlp_body,
        grid=(nb,),
        in_specs=[
            pl.BlockSpec((rb, _SD), lambda i: (i, 0)),
            pl.BlockSpec((rb, 1), lambda i: (i, 0)),
            pl.BlockSpec((rb, _SD), lambda i: (i, 0)),
            full((1, _OD)),
            full((1, _CD)),
            full((_SD + _CD, _H)),
            full((1, _H)),
            full((_H, _SD)),
            full((1, _SD)),
            full((_OD + _SD, _H)),
            full((1, _H)),
            full((_H, 1)),
            pl.BlockSpec(memory_space=pltpu.SMEM),
        ],
        out_specs=[
            pl.BlockSpec((rb, 2 * _SD), lambda i: (i, 0)),
            pl.BlockSpec((rb, 1), lambda i: (i, 0)),
        ],
        out_shape=[
            jax.ShapeDtypeStruct((_N, 2 * _SD), jnp.float32),
            jax.ShapeDtypeStruct((_N, 1), jnp.float32),
        ],
    )(sp, lw, noise, obs_r, ctrl_r, dW1, db1, dW2, db2, mW1, mb1, mW2,
      mb2)


def _finalize_call(lwp2, spred):
    return pl.pallas_call(
        _finalize_body,
        out_shape=[
            jax.ShapeDtypeStruct((8, 1024), jnp.float32),
            jax.ShapeDtypeStruct((8, 1024), jnp.float32),
            jax.ShapeDtypeStruct((1, 2 * _SD), jnp.float32),
        ],
    )(lwp2, spred)


def _sample_call(negc):
    ng = (_N - _RSC) // _ROWS
    return pl.pallas_call(
        _sample_body,
        grid=(ng,),
        in_specs=[pl.BlockSpec((8, 1024), lambda i: (0, 0))],
        out_specs=pl.BlockSpec((1, _ROWS, 1), lambda i: (i, 0, 0)),
        out_shape=jax.ShapeDtypeStruct((ng, _ROWS, 1), jnp.int32),
    )(negc)


def _sample_light_call(bits, negc):
    ng = _RSC // _ROWS
    return pl.pallas_call(
        _sample_light_body,
        grid=(ng,),
        in_specs=[pl.BlockSpec((_ROWS, _N), lambda i: (i, 0)),
                  pl.BlockSpec((8, 1024), lambda i: (0, 0))],
        out_specs=pl.BlockSpec((1, _ROWS, 1), lambda i: (i, 0, 0)),
        out_shape=jax.ShapeDtypeStruct((ng, _ROWS, 1), jnp.int32),
    )(bits, negc)


def kernel(states_prev, log_weights_prev, observation, control, dyn_W1,
           dyn_b1, dyn_W2, dyn_b2, meas_W1, meas_b1, meas_W2, meas_b2):
    noise = 0.1 * jax.random.normal(jax.random.key(42), (_N, _SD),
                                    dtype=jnp.float32)
    spred, lwp = _mlp_call(
        states_prev, log_weights_prev.reshape(_N, 1), noise,
        observation.reshape(1, _OD), control.reshape(1, _CD),
        dyn_W1, dyn_b1.reshape(1, _H), dyn_W2, dyn_b2.reshape(1, _SD),
        meas_W1, meas_b1.reshape(1, _H), meas_W2,
        meas_b2.reshape(1, 1))
    bits_sc = _sc_bits_kernel()()
    logw, negc, best = _finalize_call(lwp.reshape(8, 1024), spred)
    idx_hi = _sample_call(negc).reshape(_N - _RSC)
    idx_lo = _sample_light_call(bits_sc, negc).reshape(_RSC)
    indices = jnp.concatenate([idx_lo, idx_hi]).reshape(64, 128)
    states = _sc_gather_kernel()(spred, indices)
    return best[0, :_SD], states[:, :_SD], logw.reshape(_N)


# speedup vs baseline: 1.5393x; 1.0614x over previous
import functools

import numpy as np
import jax
import jax.numpy as jnp
from jax import lax
from jax.experimental import pallas as pl
from jax.experimental.pallas import tpu as pltpu
from jax.experimental.pallas import tpu_sc as plsc

_N = 8192
_SD = 64
_CD = 32
_OD = 128
_H = 512
_ALPHA = 0.5
_ROWS = 64
_CHUNK = 1024
_LN2 = float(np.log(2.0))
_TINY = float(np.finfo(np.float32).tiny)
_PREC = lax.Precision.HIGHEST

_NC = 2
_NS = 16
_NW = _NC * _NS
_BPW = _N // _NW

_RSC = 2304
_RPW = _RSC // _NW


def _tf_mix(x0, x1, r):
    x0 = x0 + x1
    x1 = ((x1 << r) | (x1 >> (32 - r))) ^ x0
    return x0, x1


def _threefry_0_7(n_u32):
    ks1 = jnp.uint32(7)
    ks2 = jnp.uint32(0x1BD11BDD)
    x1 = n_u32 + ks1
    x0 = x1
    x1 = ((x1 << 13) | (x1 >> 19)) ^ x0
    for r in (15, 26, 6):
        x0, x1 = _tf_mix(x0, x1, r)
    x0 = x0 + ks1
    x1 = x1 + jnp.uint32(0x1BD11BDE)
    for r in (17, 29, 16, 24):
        x0, x1 = _tf_mix(x0, x1, r)
    x0 = x0 + ks2
    x1 = x1 + jnp.uint32(2)
    for r in (13, 15, 26, 6):
        x0, x1 = _tf_mix(x0, x1, r)
    x1 = x1 + jnp.uint32(10)
    for r in (17, 29, 16, 24):
        x0, x1 = _tf_mix(x0, x1, r)
    x0 = x0 + ks1
    x1 = x1 + jnp.uint32(0x1BD11BE1)
    for r in (13, 15, 26, 6):
        x0, x1 = _tf_mix(x0, x1, r)
    x0 = x0 + ks2
    x1 = x1 + jnp.uint32(5)
    return x0 ^ x1


def _mlp_body(sp_ref, lw_ref, noise_ref, obs_ref, ctrl_ref, dW1_ref,
              db1_ref, dW2_ref, db2_ref, mW1_ref, mb1_ref, mW2_ref,
              mb2_ref, spred_ref, lwp_ref):
    sp = sp_ref[...]
    rb = sp.shape[0]
    ctrl_b = jnp.broadcast_to(ctrl_ref[...], (rb, _CD))
    h = jnp.tanh(jnp.dot(jnp.concatenate([sp, ctrl_b], axis=1),
                         dW1_ref[...]) + db1_ref[...])
    delta = jnp.dot(h, dW2_ref[...]) + db2_ref[...]
    spred = sp + delta + noise_ref[...]
    obs_b = jnp.broadcast_to(obs_ref[...], (rb, _OD))
    hm = jnp.tanh(jnp.dot(jnp.concatenate([obs_b, spred], axis=1),
                          mW1_ref[...]) + mb1_ref[...])
    ll = jnp.dot(hm, mW2_ref[...]) + mb2_ref[0, 0]
    spred_ref[...] = jnp.concatenate(
        [spred, jnp.zeros_like(spred)], axis=1)
    lwp_ref[...] = lw_ref[...] + ll


def _finalize_body(lwp_ref, spred_ref, logw_ref, negc_ref, best_ref):
    lwp = lwp_ref[...]
    w = _ALPHA * jnp.exp(lwp) + (1.0 - _ALPHA) / _N
    negc_ref[...] = -1.0 / w
    lw1 = lwp - jnp.log(w)
    amax = jnp.max(lw1)
    ls = jnp.log(jnp.sum(jnp.exp(lw1 - amax))) + amax
    logw_ref[...] = lw1 - ls
    gm = jnp.max(lwp)
    fi = (lax.broadcasted_iota(jnp.int32, (8, 1024), 0) * 1024
          + lax.broadcasted_iota(jnp.int32, (8, 1024), 1))
    bi = jnp.min(jnp.where(lwp == gm, fi, jnp.int32(_N)))
    fcol = lax.broadcasted_iota(jnp.int32, (_N, 1), 0)
    msk = (fcol == bi).astype(jnp.float32)
    best_ref[...] = jnp.sum(spred_ref[...] * msk, axis=0, keepdims=True)


@functools.cache
def _sc_gather_kernel():
    @functools.partial(
        pl.kernel,
        mesh=plsc.VectorSubcoreMesh(core_axis_name="c",
                                    subcore_axis_name="s"),
        out_type=jax.ShapeDtypeStruct((_N, 2 * _SD), jnp.float32),
        scratch_types=[
            pltpu.VMEM((2, 128), jnp.int32),
            pltpu.VMEM((_BPW, 2 * _SD), jnp.float32),
            pltpu.SemaphoreType.DMA,
        ],
    )
    def _sc_gather(table_hbm, idx_hbm, out_hbm, idx_v, rows_v, sem):
        wid = lax.axis_index("s") * _NC + lax.axis_index("c")
        base = wid * _BPW
        pltpu.sync_copy(idx_hbm.at[pl.ds(wid * 2, 2)], idx_v)
        cps = [
            pltpu.async_copy(table_hbm.at[idx_v.at[j]],
                             rows_v.at[pl.ds(j * 128, 128)], sem)
            for j in range(2)
        ]
        for cp in cps:
            cp.wait()
        pltpu.sync_copy(rows_v, out_hbm.at[pl.ds(base, _BPW)])

    return _sc_gather


def _argmin_update(met, jbase, jj, vmin, varg):
    upd = met < vmin
    vmin = jnp.where(upd, met, vmin)
    varg = jnp.where(upd, jbase + jj, varg)
    return vmin, varg


def _argmin_epilogue(vmin, varg, idx_ref):
    rowmin = jnp.min(vmin, axis=1, keepdims=True)
    cand = jnp.where(vmin == rowmin, varg, jnp.int32(_N))
    idx_ref[...] = jnp.min(cand, axis=1, keepdims=True).reshape(1, _ROWS, 1)


def _bits_to_met(bits, negc_chunk):
    fb = (bits >> jnp.uint32(9)) | jnp.uint32(0x3F800000)
    u = lax.bitcast_convert_type(fb, jnp.float32) - 1.0
    return jnp.log(u) * negc_chunk


def _sample_body(negc_ref, idx_ref):
    p = pl.program_id(0)
    di = lax.broadcasted_iota(jnp.int32, (_ROWS, _CHUNK), 0)
    jj = lax.broadcasted_iota(jnp.int32, (_ROWS, _CHUNK), 1)
    base = (_RSC + p * _ROWS + di) * _N + jj
    vmin = jnp.full((_ROWS, _CHUNK), jnp.inf, dtype=jnp.float32)
    varg = jnp.zeros((_ROWS, _CHUNK), dtype=jnp.int32)
    for cc in range(_N // _CHUNK):
        n = (base + cc * _CHUNK).astype(jnp.uint32)
        met = _bits_to_met(_threefry_0_7(n), negc_ref[cc:cc + 1, :])
        vmin, varg = _argmin_update(met, cc * _CHUNK, jj, vmin, varg)
    _argmin_epilogue(vmin, varg, idx_ref)


def _sample_light_body(bits_ref, negc_ref, idx_ref):
    jj = lax.broadcasted_iota(jnp.int32, (_ROWS, _CHUNK), 1)
    vmin = jnp.full((_ROWS, _CHUNK), jnp.inf, dtype=jnp.float32)
    varg = jnp.zeros((_ROWS, _CHUNK), dtype=jnp.int32)
    for cc in range(_N // _CHUNK):
        bits = bits_ref[:, cc * _CHUNK:(cc + 1) * _CHUNK]
        met = _bits_to_met(bits, negc_ref[cc:cc + 1, :])
        vmin, varg = _argmin_update(met, cc * _CHUNK, jj, vmin, varg)
    _argmin_epilogue(vmin, varg, idx_ref)


@functools.cache
def _sc_bits_kernel():
    @functools.partial(
        pl.kernel,
        mesh=plsc.VectorSubcoreMesh(core_axis_name="c",
                                    subcore_axis_name="s"),
        out_type=jax.ShapeDtypeStruct((_RSC, _N), jnp.uint32),
        scratch_types=[
            pltpu.VMEM((_N,), jnp.uint32),
        ],
    )
    def _sc_bits(out_hbm, buf):
        wid = lax.axis_index("s") * _NC + lax.axis_index("c")
        base_row = wid * _RPW
        lane = lax.iota(jnp.int32, 16)

        def row_body(r, carry):
            row = base_row + r

            def ck_body(ck, carry2):
                for q in range(8):
                    col = ck * 128 + q * 16
                    n = (row * _N + col + lane).astype(jnp.uint32)
                    buf[pl.ds(col, 16)] = _threefry_0_7(n)
                return carry2

            lax.fori_loop(0, _N // 128, ck_body, 0)
            pltpu.sync_copy(buf, out_hbm.at[row])
            return carry

        lax.fori_loop(0, _RPW, row_body, 0)

    return _sc_bits


def _mlp_call(sp, lw, noise, obs_r, ctrl_r, dW1, db1, dW2, db2, mW1, mb1,
              mW2, mb2):
    nb = 16
    rb = _N // nb
    full = lambda shape: pl.BlockSpec(shape, lambda i: (0, 0))
    return pl.pallas_call(
        _mlp_body,
        grid=(nb,),
        in_specs=[
            pl.BlockSpec((rb, _SD), lambda i: (i, 0)),
            pl.BlockSpec((rb, 1), lambda i: (i, 0)),
            pl.BlockSpec((rb, _SD), lambda i: (i, 0)),
            full((1, _OD)),
            full((1, _CD)),
            full((_SD + _CD, _H)),
            full((1, _H)),
            full((_H, _SD)),
            full((1, _SD)),
            full((_OD + _SD, _H)),
            full((1, _H)),
            full((_H, 1)),
            pl.BlockSpec(memory_space=pltpu.SMEM),
        ],
        out_specs=[
            pl.BlockSpec((rb, 2 * _SD), lambda i: (i, 0)),
            pl.BlockSpec((rb, 1), lambda i: (i, 0)),
        ],
        out_shape=[
            jax.ShapeDtypeStruct((_N, 2 * _SD), jnp.float32),
            jax.ShapeDtypeStruct((_N, 1), jnp.float32),
        ],
    )(sp, lw, noise, obs_r, ctrl_r, dW1, db1, dW2, db2, mW1, mb1, mW2,
      mb2)


def _finalize_call(lwp2, spred):
    return pl.pallas_call(
        _finalize_body,
        out_shape=[
            jax.ShapeDtypeStruct((8, 1024), jnp.float32),
            jax.ShapeDtypeStruct((8, 1024), jnp.float32),
            jax.ShapeDtypeStruct((1, 2 * _SD), jnp.float32),
        ],
    )(lwp2, spred)


def _sample_call(negc):
    ng = (_N - _RSC) // _ROWS
    return pl.pallas_call(
        _sample_body,
        grid=(ng,),
        in_specs=[pl.BlockSpec((8, 1024), lambda i: (0, 0))],
        out_specs=pl.BlockSpec((1, _ROWS, 1), lambda i: (i, 0, 0)),
        out_shape=jax.ShapeDtypeStruct((ng, _ROWS, 1), jnp.int32),
    )(negc)


def _sample_light_call(bits, negc):
    ng = _RSC // _ROWS
    return pl.pallas_call(
        _sample_light_body,
        grid=(ng,),
        in_specs=[pl.BlockSpec((_ROWS, _N), lambda i: (i, 0)),
                  pl.BlockSpec((8, 1024), lambda i: (0, 0))],
        out_specs=pl.BlockSpec((1, _ROWS, 1), lambda i: (i, 0, 0)),
        out_shape=jax.ShapeDtypeStruct((ng, _ROWS, 1), jnp.int32),
    )(bits, negc)


def kernel(states_prev, log_weights_prev, observation, control, dyn_W1,
           dyn_b1, dyn_W2, dyn_b2, meas_W1, meas_b1, meas_W2, meas_b2):
    noise = 0.1 * jax.random.normal(jax.random.key(42), (_N, _SD),
                                    dtype=jnp.float32)
    spred, lwp = _mlp_call(
        states_prev, log_weights_prev.reshape(_N, 1), noise,
        observation.reshape(1, _OD), control.reshape(1, _CD),
        dyn_W1, dyn_b1.reshape(1, _H), dyn_W2, dyn_b2.reshape(1, _SD),
        meas_W1, meas_b1.reshape(1, _H), meas_W2,
        meas_b2.reshape(1, 1))
    bits_sc = _sc_bits_kernel()()
    logw, negc, best = _finalize_call(lwp.reshape(8, 1024), spred)
    idx_hi = _sample_call(negc).reshape(_N - _RSC)
    idx_lo = _sample_light_call(bits_sc, negc).reshape(_RSC)
    indices = jnp.concatenate([idx_lo, idx_hi]).reshape(64, 128)
    states = _sc_gather_kernel()(spred, indices)
    return best[0, :_SD], states[:, :_SD], logw.reshape(_N)


# speedup vs baseline: 1.5719x; 1.0212x over previous
import functools

import numpy as np
import jax
import jax.numpy as jnp
from jax import lax
from jax.experimental import pallas as pl
from jax.experimental.pallas import tpu as pltpu
from jax.experimental.pallas import tpu_sc as plsc

_N = 8192
_SD = 64
_CD = 32
_OD = 128
_H = 512
_ALPHA = 0.5
_ROWS = 128
_CHUNK = 1024
_LN2 = float(np.log(2.0))
_TINY = float(np.finfo(np.float32).tiny)
_PREC = lax.Precision.HIGHEST

_NC = 2
_NS = 16
_NW = _NC * _NS
_BPW = _N // _NW

_RSC = 2304
_RPW = _RSC // _NW


def _tf_mix(x0, x1, r):
    x0 = x0 + x1
    x1 = ((x1 << r) | (x1 >> (32 - r))) ^ x0
    return x0, x1


def _threefry_0_7(n_u32):
    ks1 = jnp.uint32(7)
    ks2 = jnp.uint32(0x1BD11BDD)
    x1 = n_u32 + ks1
    x0 = x1
    x1 = ((x1 << 13) | (x1 >> 19)) ^ x0
    for r in (15, 26, 6):
        x0, x1 = _tf_mix(x0, x1, r)
    x0 = x0 + ks1
    x1 = x1 + jnp.uint32(0x1BD11BDE)
    for r in (17, 29, 16, 24):
        x0, x1 = _tf_mix(x0, x1, r)
    x0 = x0 + ks2
    x1 = x1 + jnp.uint32(2)
    for r in (13, 15, 26, 6):
        x0, x1 = _tf_mix(x0, x1, r)
    x1 = x1 + jnp.uint32(10)
    for r in (17, 29, 16, 24):
        x0, x1 = _tf_mix(x0, x1, r)
    x0 = x0 + ks1
    x1 = x1 + jnp.uint32(0x1BD11BE1)
    for r in (13, 15, 26, 6):
        x0, x1 = _tf_mix(x0, x1, r)
    x0 = x0 + ks2
    x1 = x1 + jnp.uint32(5)
    return x0 ^ x1


def _mlp_body(sp_ref, lw_ref, noise_ref, obs_ref, ctrl_ref, dW1_ref,
              db1_ref, dW2_ref, db2_ref, mW1_ref, mb1_ref, mW2_ref,
              mb2_ref, spred_ref, lwp_ref):
    sp = sp_ref[...]
    rb = sp.shape[0]
    ctrl_b = jnp.broadcast_to(ctrl_ref[...], (rb, _CD))
    h = jnp.tanh(jnp.dot(jnp.concatenate([sp, ctrl_b], axis=1),
                         dW1_ref[...]) + db1_ref[...])
    delta = jnp.dot(h, dW2_ref[...]) + db2_ref[...]
    spred = sp + delta + noise_ref[...]
    obs_b = jnp.broadcast_to(obs_ref[...], (rb, _OD))
    hm = jnp.tanh(jnp.dot(jnp.concatenate([obs_b, spred], axis=1),
                          mW1_ref[...]) + mb1_ref[...])
    ll = jnp.dot(hm, mW2_ref[...]) + mb2_ref[0, 0]
    spred_ref[...] = jnp.concatenate(
        [spred, jnp.zeros_like(spred)], axis=1)
    lwp_ref[...] = lw_ref[...] + ll


def _finalize_body(lwp_ref, spred_ref, logw_ref, negc_ref, best_ref):
    lwp = lwp_ref[...]
    w = _ALPHA * jnp.exp(lwp) + (1.0 - _ALPHA) / _N
    negc_ref[...] = -1.0 / w
    lw1 = lwp - jnp.log(w)
    amax = jnp.max(lw1)
    ls = jnp.log(jnp.sum(jnp.exp(lw1 - amax))) + amax
    logw_ref[...] = lw1 - ls
    gm = jnp.max(lwp)
    fi = (lax.broadcasted_iota(jnp.int32, (8, 1024), 0) * 1024
          + lax.broadcasted_iota(jnp.int32, (8, 1024), 1))
    bi = jnp.min(jnp.where(lwp == gm, fi, jnp.int32(_N)))
    fcol = lax.broadcasted_iota(jnp.int32, (_N, 1), 0)
    msk = (fcol == bi).astype(jnp.float32)
    best_ref[...] = jnp.sum(spred_ref[...] * msk, axis=0, keepdims=True)


@functools.cache
def _sc_gather_kernel():
    @functools.partial(
        pl.kernel,
        mesh=plsc.VectorSubcoreMesh(core_axis_name="c",
                                    subcore_axis_name="s"),
        out_type=jax.ShapeDtypeStruct((_N, 2 * _SD), jnp.float32),
        scratch_types=[
            pltpu.VMEM((2, 128), jnp.int32),
            pltpu.VMEM((_BPW, 2 * _SD), jnp.float32),
            pltpu.SemaphoreType.DMA,
        ],
    )
    def _sc_gather(table_hbm, idx_hbm, out_hbm, idx_v, rows_v, sem):
        wid = lax.axis_index("s") * _NC + lax.axis_index("c")
        base = wid * _BPW
        pltpu.sync_copy(idx_hbm.at[pl.ds(wid * 2, 2)], idx_v)
        cps = [
            pltpu.async_copy(table_hbm.at[idx_v.at[j]],
                             rows_v.at[pl.ds(j * 128, 128)], sem)
            for j in range(2)
        ]
        for cp in cps:
            cp.wait()
        pltpu.sync_copy(rows_v, out_hbm.at[pl.ds(base, _BPW)])

    return _sc_gather


def _argmin_update(met, jbase, jj, vmin, varg):
    upd = met < vmin
    vmin = jnp.where(upd, met, vmin)
    varg = jnp.where(upd, jbase + jj, varg)
    return vmin, varg


def _argmin_epilogue(vmin, varg, idx_ref):
    rowmin = jnp.min(vmin, axis=1, keepdims=True)
    cand = jnp.where(vmin == rowmin, varg, jnp.int32(_N))
    idx_ref[...] = jnp.min(cand, axis=1, keepdims=True).reshape(1, _ROWS, 1)


def _bits_to_met(bits, negc_chunk):
    fb = (bits >> jnp.uint32(9)) | jnp.uint32(0x3F800000)
    u = lax.bitcast_convert_type(fb, jnp.float32) - 1.0
    return jnp.log(u) * negc_chunk


def _sample_body(negc_ref, idx_ref):
    p = pl.program_id(0)
    di = lax.broadcasted_iota(jnp.int32, (_ROWS, _CHUNK), 0)
    jj = lax.broadcasted_iota(jnp.int32, (_ROWS, _CHUNK), 1)
    base = (_RSC + p * _ROWS + di) * _N + jj
    vmin = jnp.full((_ROWS, _CHUNK), jnp.inf, dtype=jnp.float32)
    varg = jnp.zeros((_ROWS, _CHUNK), dtype=jnp.int32)
    for cc in range(_N // _CHUNK):
        n = (base + cc * _CHUNK).astype(jnp.uint32)
        met = _bits_to_met(_threefry_0_7(n), negc_ref[cc:cc + 1, :])
        vmin, varg = _argmin_update(met, cc * _CHUNK, jj, vmin, varg)
    _argmin_epilogue(vmin, varg, idx_ref)


def _sample_light_body(bits_ref, negc_ref, idx_ref):
    jj = lax.broadcasted_iota(jnp.int32, (_ROWS, _CHUNK), 1)
    vmin = jnp.full((_ROWS, _CHUNK), jnp.inf, dtype=jnp.float32)
    varg = jnp.zeros((_ROWS, _CHUNK), dtype=jnp.int32)
    for cc in range(_N // _CHUNK):
        bits = bits_ref[:, cc * _CHUNK:(cc + 1) * _CHUNK]
        met = _bits_to_met(bits, negc_ref[cc:cc + 1, :])
        vmin, varg = _argmin_update(met, cc * _CHUNK, jj, vmin, varg)
    _argmin_epilogue(vmin, varg, idx_ref)


@functools.cache
def _sc_bits_kernel():
    @functools.partial(
        pl.kernel,
        mesh=plsc.VectorSubcoreMesh(core_axis_name="c",
                                    subcore_axis_name="s"),
        out_type=jax.ShapeDtypeStruct((_RSC, _N), jnp.uint32),
        scratch_types=[
            pltpu.VMEM((_N,), jnp.uint32),
        ],
    )
    def _sc_bits(out_hbm, buf):
        wid = lax.axis_index("s") * _NC + lax.axis_index("c")
        base_row = wid * _RPW
        lane = lax.iota(jnp.int32, 16)

        def row_body(r, carry):
            row = base_row + r

            def ck_body(ck, carry2):
                for q in range(8):
                    col = ck * 128 + q * 16
                    n = (row * _N + col + lane).astype(jnp.uint32)
                    buf[pl.ds(col, 16)] = _threefry_0_7(n)
                return carry2

            lax.fori_loop(0, _N // 128, ck_body, 0)
            pltpu.sync_copy(buf, out_hbm.at[row])
            return carry

        lax.fori_loop(0, _RPW, row_body, 0)

    return _sc_bits


def _mlp_call(sp, lw, noise, obs_r, ctrl_r, dW1, db1, dW2, db2, mW1, mb1,
              mW2, mb2):
    nb = 16
    rb = _N // nb
    full = lambda shape: pl.BlockSpec(shape, lambda i: (0, 0))
    return pl.pallas_call(
        _mlp_body,
        grid=(nb,),
        in_specs=[
            pl.BlockSpec((rb, _SD), lambda i: (i, 0)),
            pl.BlockSpec((rb, 1), lambda i: (i, 0)),
            pl.BlockSpec((rb, _SD), lambda i: (i, 0)),
            full((1, _OD)),
            full((1, _CD)),
            full((_SD + _CD, _H)),
            full((1, _H)),
            full((_H, _SD)),
            full((1, _SD)),
            full((_OD + _SD, _H)),
            full((1, _H)),
            full((_H, 1)),
            pl.BlockSpec(memory_space=pltpu.SMEM),
        ],
        out_specs=[
            pl.BlockSpec((rb, 2 * _SD), lambda i: (i, 0)),
            pl.BlockSpec((rb, 1), lambda i: (i, 0)),
        ],
        out_shape=[
            jax.ShapeDtypeStruct((_N, 2 * _SD), jnp.float32),
            jax.ShapeDtypeStruct((_N, 1), jnp.float32),
        ],
    )(sp, lw, noise, obs_r, ctrl_r, dW1, db1, dW2, db2, mW1, mb1, mW2,
      mb2)


def _finalize_call(lwp2, spred):
    return pl.pallas_call(
        _finalize_body,
        out_shape=[
            jax.ShapeDtypeStruct((8, 1024), jnp.float32),
            jax.ShapeDtypeStruct((8, 1024), jnp.float32),
            jax.ShapeDtypeStruct((1, 2 * _SD), jnp.float32),
        ],
    )(lwp2, spred)


def _sample_call(negc):
    ng = (_N - _RSC) // _ROWS
    return pl.pallas_call(
        _sample_body,
        grid=(ng,),
        in_specs=[pl.BlockSpec((8, 1024), lambda i: (0, 0))],
        out_specs=pl.BlockSpec((1, _ROWS, 1), lambda i: (i, 0, 0)),
        out_shape=jax.ShapeDtypeStruct((ng, _ROWS, 1), jnp.int32),
    )(negc)


def _sample_light_call(bits, negc):
    ng = _RSC // _ROWS
    return pl.pallas_call(
        _sample_light_body,
        grid=(ng,),
        in_specs=[pl.BlockSpec((_ROWS, _N), lambda i: (i, 0)),
                  pl.BlockSpec((8, 1024), lambda i: (0, 0))],
        out_specs=pl.BlockSpec((1, _ROWS, 1), lambda i: (i, 0, 0)),
        out_shape=jax.ShapeDtypeStruct((ng, _ROWS, 1), jnp.int32),
    )(bits, negc)


def kernel(states_prev, log_weights_prev, observation, control, dyn_W1,
           dyn_b1, dyn_W2, dyn_b2, meas_W1, meas_b1, meas_W2, meas_b2):
    noise = 0.1 * jax.random.normal(jax.random.key(42), (_N, _SD),
                                    dtype=jnp.float32)
    spred, lwp = _mlp_call(
        states_prev, log_weights_prev.reshape(_N, 1), noise,
        observation.reshape(1, _OD), control.reshape(1, _CD),
        dyn_W1, dyn_b1.reshape(1, _H), dyn_W2, dyn_b2.reshape(1, _SD),
        meas_W1, meas_b1.reshape(1, _H), meas_W2,
        meas_b2.reshape(1, 1))
    bits_sc = _sc_bits_kernel()()
    logw, negc, best = _finalize_call(lwp.reshape(8, 1024), spred)
    idx_hi = _sample_call(negc).reshape(_N - _RSC)
    idx_lo = _sample_light_call(bits_sc, negc).reshape(_RSC)
    indices = jnp.concatenate([idx_lo, idx_hi]).reshape(64, 128)
    states = _sc_gather_kernel()(spred, indices)
    return best[0, :_SD], states[:, :_SD], logw.reshape(_N)


# speedup vs baseline: 1.5887x; 1.0107x over previous
import functools

import numpy as np
import jax
import jax.numpy as jnp
from jax import lax
from jax.experimental import pallas as pl
from jax.experimental.pallas import tpu as pltpu
from jax.experimental.pallas import tpu_sc as plsc

_N = 8192
_SD = 64
_CD = 32
_OD = 128
_H = 512
_ALPHA = 0.5
_ROWS = 128
_CHUNK = 1024
_LN2 = float(np.log(2.0))
_TINY = float(np.finfo(np.float32).tiny)
_PREC = lax.Precision.HIGHEST

_NC = 2
_NS = 16
_NW = _NC * _NS
_BPW = _N // _NW

_RSC = 2304
_RPW = _RSC // _NW


def _tf_mix(x0, x1, r):
    x0 = x0 + x1
    x1 = ((x1 << r) | (x1 >> (32 - r))) ^ x0
    return x0, x1


def _threefry_key(n_u32, seed):
    k1 = seed & 0xFFFFFFFF
    k2 = (k1 ^ 0x1BD11BDA) & 0xFFFFFFFF
    ks1 = jnp.uint32(k1)
    ks2 = jnp.uint32(k2)
    x1 = n_u32 + ks1
    x0 = x1
    x1 = ((x1 << 13) | (x1 >> 19)) ^ x0
    for r in (15, 26, 6):
        x0, x1 = _tf_mix(x0, x1, r)
    x0 = x0 + ks1
    x1 = x1 + jnp.uint32((k2 + 1) & 0xFFFFFFFF)
    for r in (17, 29, 16, 24):
        x0, x1 = _tf_mix(x0, x1, r)
    x0 = x0 + ks2
    x1 = x1 + jnp.uint32(2)
    for r in (13, 15, 26, 6):
        x0, x1 = _tf_mix(x0, x1, r)
    x1 = x1 + jnp.uint32((k1 + 3) & 0xFFFFFFFF)
    for r in (17, 29, 16, 24):
        x0, x1 = _tf_mix(x0, x1, r)
    x0 = x0 + ks1
    x1 = x1 + jnp.uint32((k2 + 4) & 0xFFFFFFFF)
    for r in (13, 15, 26, 6):
        x0, x1 = _tf_mix(x0, x1, r)
    x0 = x0 + ks2
    x1 = x1 + jnp.uint32(5)
    return x0 ^ x1


def _threefry_0_7(n_u32):
    return _threefry_key(n_u32, 7)


def _mlp_body(sp_ref, lw_ref, obs_ref, ctrl_ref, dW1_ref,
              db1_ref, dW2_ref, db2_ref, mW1_ref, mb1_ref, mW2_ref,
              mb2_ref, spred_ref, lwp_ref):
    sp = sp_ref[...]
    rb = sp.shape[0]
    ri = lax.broadcasted_iota(jnp.int32, (rb, _SD), 0)
    ci = lax.broadcasted_iota(jnp.int32, (rb, _SD), 1)
    nn = ((pl.program_id(0) * rb + ri) * _SD + ci).astype(jnp.uint32)
    nb = _threefry_key(nn, 42)
    nf = lax.bitcast_convert_type(
        (nb >> jnp.uint32(9)) | jnp.uint32(0x3F800000), jnp.float32) - 1.0
    _LO = -0.9999999403953552
    un = jnp.maximum(nf * 2.0 + _LO, _LO)
    noise = 0.1 * (1.4142135623730951 * lax.erf_inv(un))
    ctrl_b = jnp.broadcast_to(ctrl_ref[...], (rb, _CD))
    h = jnp.tanh(jnp.dot(jnp.concatenate([sp, ctrl_b], axis=1),
                         dW1_ref[...]) + db1_ref[...])
    delta = jnp.dot(h, dW2_ref[...]) + db2_ref[...]
    spred = sp + delta + noise
    obs_b = jnp.broadcast_to(obs_ref[...], (rb, _OD))
    hm = jnp.tanh(jnp.dot(jnp.concatenate([obs_b, spred], axis=1),
                          mW1_ref[...]) + mb1_ref[...])
    ll = jnp.dot(hm, mW2_ref[...]) + mb2_ref[0, 0]
    spred_ref[...] = jnp.concatenate(
        [spred, jnp.zeros_like(spred)], axis=1)
    lwp_ref[...] = lw_ref[...] + ll


def _finalize_body(lwp_ref, spred_ref, logw_ref, negc_ref, best_ref):
    lwp = lwp_ref[...]
    w = _ALPHA * jnp.exp(lwp) + (1.0 - _ALPHA) / _N
    negc_ref[...] = -1.0 / w
    lw1 = lwp - jnp.log(w)
    amax = jnp.max(lw1)
    ls = jnp.log(jnp.sum(jnp.exp(lw1 - amax))) + amax
    logw_ref[...] = lw1 - ls
    gm = jnp.max(lwp)
    fi = (lax.broadcasted_iota(jnp.int32, (8, 1024), 0) * 1024
          + lax.broadcasted_iota(jnp.int32, (8, 1024), 1))
    bi = jnp.min(jnp.where(lwp == gm, fi, jnp.int32(_N)))
    fcol = lax.broadcasted_iota(jnp.int32, (_N, 1), 0)
    msk = (fcol == bi).astype(jnp.float32)
    best_ref[...] = jnp.sum(spred_ref[...] * msk, axis=0, keepdims=True)


@functools.cache
def _sc_gather_kernel():
    @functools.partial(
        pl.kernel,
        mesh=plsc.VectorSubcoreMesh(core_axis_name="c",
                                    subcore_axis_name="s"),
        out_type=jax.ShapeDtypeStruct((_N, 2 * _SD), jnp.float32),
        scratch_types=[
            pltpu.VMEM((2, 128), jnp.int32),
            pltpu.VMEM((_BPW, 2 * _SD), jnp.float32),
            pltpu.SemaphoreType.DMA,
        ],
    )
    def _sc_gather(table_hbm, idx_hbm, out_hbm, idx_v, rows_v, sem):
        wid = lax.axis_index("s") * _NC + lax.axis_index("c")
        base = wid * _BPW
        pltpu.sync_copy(idx_hbm.at[pl.ds(wid * 2, 2)], idx_v)
        cps = [
            pltpu.async_copy(table_hbm.at[idx_v.at[j]],
                             rows_v.at[pl.ds(j * 128, 128)], sem)
            for j in range(2)
        ]
        for cp in cps:
            cp.wait()
        pltpu.sync_copy(rows_v, out_hbm.at[pl.ds(base, _BPW)])

    return _sc_gather


def _argmin_update(met, jbase, jj, vmin, varg):
    upd = met < vmin
    vmin = jnp.where(upd, met, vmin)
    varg = jnp.where(upd, jbase + jj, varg)
    return vmin, varg


def _argmin_epilogue(vmin, varg, idx_ref):
    rowmin = jnp.min(vmin, axis=1, keepdims=True)
    cand = jnp.where(vmin == rowmin, varg, jnp.int32(_N))
    idx_ref[...] = jnp.min(cand, axis=1, keepdims=True).reshape(1, _ROWS, 1)


def _bits_to_met(bits, negc_chunk):
    fb = (bits >> jnp.uint32(9)) | jnp.uint32(0x3F800000)
    u = lax.bitcast_convert_type(fb, jnp.float32) - 1.0
    return jnp.log(u) * negc_chunk


def _sample_body(negc_ref, idx_ref):
    p = pl.program_id(0)
    di = lax.broadcasted_iota(jnp.int32, (_ROWS, _CHUNK), 0)
    jj = lax.broadcasted_iota(jnp.int32, (_ROWS, _CHUNK), 1)
    base = (_RSC + p * _ROWS + di) * _N + jj
    vmin = jnp.full((_ROWS, _CHUNK), jnp.inf, dtype=jnp.float32)
    varg = jnp.zeros((_ROWS, _CHUNK), dtype=jnp.int32)
    for cc in range(_N // _CHUNK):
        n = (base + cc * _CHUNK).astype(jnp.uint32)
        met = _bits_to_met(_threefry_0_7(n), negc_ref[cc:cc + 1, :])
        vmin, varg = _argmin_update(met, cc * _CHUNK, jj, vmin, varg)
    _argmin_epilogue(vmin, varg, idx_ref)


def _sample_light_body(bits_ref, negc_ref, idx_ref):
    jj = lax.broadcasted_iota(jnp.int32, (_ROWS, _CHUNK), 1)
    vmin = jnp.full((_ROWS, _CHUNK), jnp.inf, dtype=jnp.float32)
    varg = jnp.zeros((_ROWS, _CHUNK), dtype=jnp.int32)
    for cc in range(_N // _CHUNK):
        bits = bits_ref[:, cc * _CHUNK:(cc + 1) * _CHUNK]
        met = _bits_to_met(bits, negc_ref[cc:cc + 1, :])
        vmin, varg = _argmin_update(met, cc * _CHUNK, jj, vmin, varg)
    _argmin_epilogue(vmin, varg, idx_ref)


@functools.cache
def _sc_bits_kernel():
    @functools.partial(
        pl.kernel,
        mesh=plsc.VectorSubcoreMesh(core_axis_name="c",
                                    subcore_axis_name="s"),
        out_type=jax.ShapeDtypeStruct((_RSC, _N), jnp.uint32),
        scratch_types=[
            pltpu.VMEM((_N,), jnp.uint32),
        ],
    )
    def _sc_bits(out_hbm, buf):
        wid = lax.axis_index("s") * _NC + lax.axis_index("c")
        base_row = wid * _RPW
        lane = lax.iota(jnp.int32, 16)

        def row_body(r, carry):
            row = base_row + r

            def ck_body(ck, carry2):
                for q in range(8):
                    col = ck * 128 + q * 16
                    n = (row * _N + col + lane).astype(jnp.uint32)
                    buf[pl.ds(col, 16)] = _threefry_0_7(n)
                return carry2

            lax.fori_loop(0, _N // 128, ck_body, 0)
            pltpu.sync_copy(buf, out_hbm.at[row])
            return carry

        lax.fori_loop(0, _RPW, row_body, 0)

    return _sc_bits


def _mlp_call(sp, lw, obs_r, ctrl_r, dW1, db1, dW2, db2, mW1, mb1,
              mW2, mb2):
    nb = 16
    rb = _N // nb
    full = lambda shape: pl.BlockSpec(shape, lambda i: (0, 0))
    return pl.pallas_call(
        _mlp_body,
        grid=(nb,),
        in_specs=[
            pl.BlockSpec((rb, _SD), lambda i: (i, 0)),
            pl.BlockSpec((rb, 1), lambda i: (i, 0)),
            full((1, _OD)),
            full((1, _CD)),
            full((_SD + _CD, _H)),
            full((1, _H)),
            full((_H, _SD)),
            full((1, _SD)),
            full((_OD + _SD, _H)),
            full((1, _H)),
            full((_H, 1)),
            pl.BlockSpec(memory_space=pltpu.SMEM),
        ],
        out_specs=[
            pl.BlockSpec((rb, 2 * _SD), lambda i: (i, 0)),
            pl.BlockSpec((rb, 1), lambda i: (i, 0)),
        ],
        out_shape=[
            jax.ShapeDtypeStruct((_N, 2 * _SD), jnp.float32),
            jax.ShapeDtypeStruct((_N, 1), jnp.float32),
        ],
    )(sp, lw, obs_r, ctrl_r, dW1, db1, dW2, db2, mW1, mb1, mW2, mb2)


def _finalize_call(lwp2, spred):
    return pl.pallas_call(
        _finalize_body,
        out_shape=[
            jax.ShapeDtypeStruct((8, 1024), jnp.float32),
            jax.ShapeDtypeStruct((8, 1024), jnp.float32),
            jax.ShapeDtypeStruct((1, 2 * _SD), jnp.float32),
        ],
    )(lwp2, spred)


def _sample_call(negc):
    ng = (_N - _RSC) // _ROWS
    return pl.pallas_call(
        _sample_body,
        grid=(ng,),
        in_specs=[pl.BlockSpec((8, 1024), lambda i: (0, 0))],
        out_specs=pl.BlockSpec((1, _ROWS, 1), lambda i: (i, 0, 0)),
        out_shape=jax.ShapeDtypeStruct((ng, _ROWS, 1), jnp.int32),
    )(negc)


def _sample_light_call(bits, negc):
    ng = _RSC // _ROWS
    return pl.pallas_call(
        _sample_light_body,
        grid=(ng,),
        in_specs=[pl.BlockSpec((_ROWS, _N), lambda i: (i, 0)),
                  pl.BlockSpec((8, 1024), lambda i: (0, 0))],
        out_specs=pl.BlockSpec((1, _ROWS, 1), lambda i: (i, 0, 0)),
        out_shape=jax.ShapeDtypeStruct((ng, _ROWS, 1), jnp.int32),
    )(bits, negc)


def kernel(states_prev, log_weights_prev, observation, control, dyn_W1,
           dyn_b1, dyn_W2, dyn_b2, meas_W1, meas_b1, meas_W2, meas_b2):
    spred, lwp = _mlp_call(
        states_prev, log_weights_prev.reshape(_N, 1),
        observation.reshape(1, _OD), control.reshape(1, _CD),
        dyn_W1, dyn_b1.reshape(1, _H), dyn_W2, dyn_b2.reshape(1, _SD),
        meas_W1, meas_b1.reshape(1, _H), meas_W2,
        meas_b2.reshape(1, 1))
    bits_sc = _sc_bits_kernel()()
    logw, negc, best = _finalize_call(lwp.reshape(8, 1024), spred)
    idx_hi = _sample_call(negc).reshape(_N - _RSC)
    idx_lo = _sample_light_call(bits_sc, negc).reshape(_RSC)
    indices = jnp.concatenate([idx_lo, idx_hi]).reshape(64, 128)
    states = _sc_gather_kernel()(spred, indices)
    return best[0, :_SD], states[:, :_SD], logw.reshape(_N)
